# Initial kernel scaffold; baseline (speedup 1.0000x reference)
import jax
import jax.numpy as jnp
from jax.experimental import pallas as pl


def kernel(x, target, mu, logvar):
    raise NotImplementedError("write your pallas kernel here")


# speedup vs baseline: 6.1051x; 6.1051x over previous
import functools

import jax
import jax.numpy as jnp
from jax import lax
from jax.experimental import pallas as pl
from jax.experimental.pallas import tpu as pltpu
import numpy as np

_SIZE = 32000
_PAD = 0
_SMOOTHING = 0.1
_CONFIDENCE = 1.0 - _SMOOTHING
_N_TOK = 4096
_LATENT = 512

_EPS = np.float32(_SMOOTHING / (_SIZE - 2))
_YLOGY = np.float32(
    (_SIZE - 2) * float(_EPS) * np.log(float(_EPS))
    + _CONFIDENCE * np.log(_CONFIDENCE)
)

_RBLK = 1024
_CBLK = 1280
_RGRID = _N_TOK // _RBLK
_CGRID = _SIZE // _CBLK


def _loss_body(x_ref, tgt_ref, mu_ref, lv_ref, rec_ref, kl_ref, acc_ref):
    i = pl.program_id(0)
    j = pl.program_id(1)

    @pl.when((i == 0) & (j == 0))
    def _init():
        acc_ref[0] = 0.0
        acc_ref[1] = 0.0
        acc_ref[2] = 0.0
        acc_ref[3] = 0.0
        acc_ref[4] = 0.0
        rec_ref[0, 0] = 0.0
        kl_ref[0, 0] = 0.0

    tgt = tgt_ref[0, 0, :]
    w = (tgt != _PAD).astype(jnp.float32)
    xw = x_ref[...] * w[:, None]

    acc_ref[0] = acc_ref[0] + jnp.sum(xw)
    col = j * _CBLK + lax.broadcasted_iota(jnp.int32, (_RBLK, _CBLK), 1)
    hit = col == tgt[:, None]
    acc_ref[1] = acc_ref[1] + jnp.sum(jnp.where(hit, xw, 0.0))

    @pl.when(j == 0)
    def _once_per_rowblock():
        acc_ref[2] = acc_ref[2] + jnp.sum(xw[:, _PAD])
        acc_ref[3] = acc_ref[3] + jnp.sum(w)
        lv = lv_ref[...]
        mu = mu_ref[...]
        acc_ref[4] = acc_ref[4] + jnp.sum(1.0 + lv - mu * mu - jnp.exp(lv))

    @pl.when((i == _RGRID - 1) & (j == _CGRID - 1))
    def _finalize():
        rec_ref[0, 0] = (
            acc_ref[3] * _YLOGY
            - _EPS * (acc_ref[0] - acc_ref[2])
            - (np.float32(_CONFIDENCE) - _EPS) * acc_ref[1]
        )
        kl_ref[0, 0] = -0.5 * acc_ref[4] / np.float32(_N_TOK * _LATENT)


@jax.jit
def kernel(x, target, mu, logvar):
    tgt3 = target.reshape(_RGRID, 1, _RBLK)
    rec, kl = pl.pallas_call(
        _loss_body,
        grid=(_RGRID, _CGRID),
        in_specs=[
            pl.BlockSpec((_RBLK, _CBLK), lambda i, j: (i, j)),
            pl.BlockSpec((1, 1, _RBLK), lambda i, j: (i, 0, 0)),
            pl.BlockSpec((_RBLK, _LATENT), lambda i, j: (i, 0)),
            pl.BlockSpec((_RBLK, _LATENT), lambda i, j: (i, 0)),
        ],
        out_specs=[
            pl.BlockSpec(memory_space=pltpu.SMEM),
            pl.BlockSpec(memory_space=pltpu.SMEM),
        ],
        out_shape=[
            jax.ShapeDtypeStruct((1, 1), jnp.float32),
            jax.ShapeDtypeStruct((1, 1), jnp.float32),
        ],
        scratch_shapes=[pltpu.SMEM((5,), jnp.float32)],
    )(x, tgt3, mu, logvar)
    return (rec[0, 0], kl[0, 0])
